# Initial kernel scaffold; baseline (speedup 1.0000x reference)
import jax
import jax.numpy as jnp
from jax.experimental import pallas as pl


def kernel(pred_logits, pred_boxes, pred_vectors, target_sizes):
    raise NotImplementedError("write your pallas kernel here")


# speedup vs baseline: 3.7406x; 3.7406x over previous
import functools

import jax
import jax.numpy as jnp
from jax import lax
from jax.experimental import pallas as pl
from jax.experimental.pallas import tpu as pltpu
from jax.experimental.pallas import tpu_sc as plsc

B, Q, C, K = 64, 900, 91, 50
QC = Q * C
BLK = 256
QCP = 81920
NBLK = QCP // BLK
KP = 64
NEG_BIG = 1 << 30


def _shuf(x, iota, sft):
    return x.at[iota ^ sft].get(mode="promise_in_bounds")


def _hmax(x, iota):
    for sft in (1, 2, 4, 8):
        x = jnp.maximum(x, _shuf(x, iota, sft))
    return x


def _hmin(x, iota):
    for sft in (1, 2, 4, 8):
        x = jnp.minimum(x, _shuf(x, iota, sft))
    return x


def _tec_body(prob_hbm, boxes_hbm, ts_hbm,
              scores_hbm, labels_hbm, boxeso_hbm,
              data_v, boxes_v, m_v, vals_v, idxs_v, ts_v, lab_v, box_v):
    c = lax.axis_index("c")
    s = lax.axis_index("s")
    wid = s * 2 + c
    iota = lax.iota(jnp.int32, 16)
    zero16 = jnp.zeros((16,), jnp.int32)

    for g in range(4):
        idxs_v[pl.ds(g * 16, 16)] = zero16

    pltpu.sync_copy(ts_hbm, ts_v)

    for r in range(2):
        b = wid * 2 + r
        pltpu.sync_copy(prob_hbm.at[b], data_v)
        pltpu.sync_copy(boxes_hbm.at[b], boxes_v)

        def blk_body(i, _):
            acc = data_v[pl.ds(i * BLK, 16)]
            for t in range(1, 16):
                acc = jnp.maximum(acc, data_v[pl.ds(i * BLK + t * 16, 16)])
            mx = _hmax(acc, iota)
            ilane = i & 15
            ioff = i - ilane
            mvec = m_v[pl.ds(ioff, 16)]
            m_v[pl.ds(ioff, 16)] = jnp.where(iota == ilane, mx, mvec)
            return 0

        lax.fori_loop(0, NBLK, blk_body, 0)

        def extract(k, _):
            def scanm(j, carry):
                accc, bidx = carry
                v = m_v[pl.ds(j * 16, 16)]
                m = v > accc
                return (jnp.where(m, v, accc),
                        jnp.where(m, j * 16 + iota, bidx))

            acc, bidx = lax.fori_loop(
                0, NBLK // 16, scanm,
                (jnp.full((16,), -3.0, jnp.float32), zero16))
            gvec = _hmax(acc, iota)
            bstar = _hmin(jnp.where(acc == gvec, bidx, NEG_BIG), iota)[0]

            def scanb(t, carry):
                acc2, eidx = carry
                base = bstar * BLK + t * 16
                v = data_v[pl.ds(base, 16)]
                m = v > acc2
                return (jnp.where(m, v, acc2),
                        jnp.where(m, base + iota, eidx))

            acc2, eidx = lax.fori_loop(
                0, 16, scanb,
                (jnp.full((16,), -3.0, jnp.float32), zero16))
            estar = _hmin(jnp.where(acc2 == gvec, eidx, NEG_BIG), iota)[0]

            lane = k & 15
            off = k - lane
            vv = vals_v[pl.ds(off, 16)]
            vals_v[pl.ds(off, 16)] = jnp.where(iota == lane, gvec, vv)
            iv = idxs_v[pl.ds(off, 16)]
            idxs_v[pl.ds(off, 16)] = jnp.where(iota == lane, estar, iv)

            el = estar & 15
            eoff = estar - el
            dv = data_v[pl.ds(eoff, 16)]
            data_v[pl.ds(eoff, 16)] = jnp.where(iota == el, jnp.float32(-2.0), dv)

            acc3 = data_v[pl.ds(bstar * BLK, 16)]
            for t in range(1, 16):
                acc3 = jnp.maximum(acc3, data_v[pl.ds(bstar * BLK + t * 16, 16)])
            nm = _hmax(acc3, iota)
            bl = bstar & 15
            boff = bstar - bl
            mv = m_v[pl.ds(boff, 16)]
            m_v[pl.ds(boff, 16)] = jnp.where(iota == bl, nm, mv)
            return 0

        lax.fori_loop(0, K, extract, 0)

        img_h = plsc.load_gather(ts_v, [jnp.full((16,), 2 * b, jnp.int32)])
        img_w = plsc.load_gather(ts_v, [jnp.full((16,), 2 * b + 1, jnp.int32)])
        for g in range(4):
            idxv = idxs_v[pl.ds(g * 16, 16)]
            fidx = idxv.astype(jnp.float32)
            q = ((fidx + 0.5) / jnp.float32(C)).astype(jnp.int32)
            labv = idxv - q * C
            lab_v[pl.ds(g * 16, 16)] = labv
            q4 = q * 4
            cx = plsc.load_gather(boxes_v, [q4])
            cy = plsc.load_gather(boxes_v, [q4 + 1])
            w_ = plsc.load_gather(boxes_v, [q4 + 2])
            h_ = plsc.load_gather(boxes_v, [q4 + 3])
            x0 = (cx - 0.5 * w_) * img_w
            y0 = (cy - 0.5 * h_) * img_h
            x1 = (cx + 0.5 * w_) * img_w
            y1 = (cy + 0.5 * h_) * img_h
            k4 = (g * 16 + iota) * 4
            plsc.store_scatter(box_v, [k4], x0)
            plsc.store_scatter(box_v, [k4 + 1], y0)
            plsc.store_scatter(box_v, [k4 + 2], x1)
            plsc.store_scatter(box_v, [k4 + 3], y1)

        pltpu.sync_copy(vals_v, scores_hbm.at[b])
        pltpu.sync_copy(lab_v, labels_hbm.at[b])
        pltpu.sync_copy(box_v, boxeso_hbm.at[b])


_sc_call = functools.partial(
    pl.kernel,
    out_type=[
        jax.ShapeDtypeStruct((B, KP), jnp.float32),
        jax.ShapeDtypeStruct((B, KP), jnp.int32),
        jax.ShapeDtypeStruct((B, KP * 4), jnp.float32),
    ],
    mesh=plsc.VectorSubcoreMesh(core_axis_name="c", subcore_axis_name="s"),
    compiler_params=pltpu.CompilerParams(needs_layout_passes=False),
    scratch_types=[
        pltpu.VMEM((QCP,), jnp.float32),
        pltpu.VMEM((Q * 4,), jnp.float32),
        pltpu.VMEM((NBLK,), jnp.float32),
        pltpu.VMEM((KP,), jnp.float32),
        pltpu.VMEM((KP,), jnp.int32),
        pltpu.VMEM((B * 2,), jnp.float32),
        pltpu.VMEM((KP,), jnp.int32),
        pltpu.VMEM((KP * 4,), jnp.float32),
    ],
)(_tec_body)


def kernel(pred_logits, pred_boxes, pred_vectors, target_sizes):
    del pred_vectors
    prob = jax.nn.sigmoid(pred_logits).reshape(B, QC)
    probp = jnp.pad(prob, ((0, 0), (0, QCP - QC)), constant_values=-1.0)
    scores, labels, boxes = _sc_call(
        probp, pred_boxes.reshape(B, Q * 4), target_sizes.reshape(B * 2))
    return scores[:, :K], labels[:, :K], boxes.reshape(B, KP, 4)[:, :K, :]
